# Initial kernel scaffold; baseline (speedup 1.0000x reference)
import jax
import jax.numpy as jnp
from jax.experimental import pallas as pl


def kernel(x, edge_index, edge_type, W1, root1, b1, W2, root2, b2):
    raise NotImplementedError("write your pallas kernel here")


# speedup vs baseline: 9.4035x; 9.4035x over previous
import functools

import jax
import jax.numpy as jnp
from jax import lax
from jax.experimental import pallas as pl
from jax.experimental.pallas import tpu as pltpu
from jax.experimental.pallas import tpu_sc as plsc

N = 10000
E = 320000
R = 4
CH = 128
OUT = 2
CNTP = 40960

NC = 2
NS = 16
NW = NC * NS
EPW = E // NW
CHUNK = 80
NCHUNK = EPW // CHUNK
EPC = E // NS
NCCHUNK = EPC // CHUNK
NP = 10240
RPT = NP // NS
CSTRIPE = CNTP // NS

_MESH = dict(core_axis_name="c", subcore_axis_name="s")

BLK = 400
NBLK = N // BLK


@functools.partial(
    pl.kernel,
    out_type=jax.ShapeDtypeStruct((E,), jnp.float32),
    mesh=plsc.VectorSubcoreMesh(**_MESH),
    scratch_types=[
        pltpu.VMEM((CHUNK + 16,), jnp.int32),
        pltpu.VMEM((CHUNK,), jnp.float32),
        pltpu.VMEM((CHUNK,), jnp.float32),
        pltpu.VMEM((CSTRIPE,), jnp.float32),
        pltpu.VMEM((CNTP + 16,), jnp.float32),
        pltpu.VMEM_SHARED((CNTP,), jnp.float32),
    ],
)
def _sc_scales(sidx_hbm, scl_hbm, six_v, ones_v, s_v, stripe_v, inv_v, cnt_sh):
    c = lax.axis_index("c")
    s = lax.axis_index("s")
    wid = s * NC + c

    def _fill_ones(i, _):
        ones_v[pl.ds(i * 16, 16)] = jnp.full((16,), 1.0, jnp.float32)
        return 0
    lax.fori_loop(0, CHUNK // 16, _fill_ones, 0)

    def _fill_zeros(i, _):
        stripe_v[pl.ds(i * 16, 16)] = jnp.zeros((16,), jnp.float32)
        return 0
    lax.fori_loop(0, CSTRIPE // 16, _fill_zeros, 0)

    pltpu.sync_copy(stripe_v, cnt_sh.at[pl.ds(s * CSTRIPE, CSTRIPE)])
    plsc.subcore_barrier()

    cbase = s * EPC

    def _count(k, _):
        pltpu.sync_copy(sidx_hbm.at[pl.ds(cbase + k * CHUNK, CHUNK)],
                        six_v.at[pl.ds(0, CHUNK)])
        pltpu.sync_copy(ones_v, cnt_sh.at[six_v.at[pl.ds(0, CHUNK)]],
                        add=True)
        return 0
    lax.fori_loop(0, NCCHUNK, _count, 0)
    plsc.subcore_barrier()

    pltpu.sync_copy(cnt_sh.at[pl.ds(s * CSTRIPE, CSTRIPE)], stripe_v)

    def _invb(i, _):
        v = stripe_v[pl.ds(i * 16, 16)]
        stripe_v[pl.ds(i * 16, 16)] = 1.0 / jnp.maximum(v, 1.0)
        return 0
    lax.fori_loop(0, CSTRIPE // 16, _invb, 0)
    pltpu.sync_copy(stripe_v, cnt_sh.at[pl.ds(s * CSTRIPE, CSTRIPE)])
    plsc.subcore_barrier()

    pltpu.sync_copy(cnt_sh, inv_v.at[pl.ds(0, CNTP)])
    lanes = lax.iota(jnp.int32, 16)
    base = wid * EPW

    def _extract(k, _):
        off = base + k * CHUNK
        pltpu.sync_copy(sidx_hbm.at[pl.ds(off, CHUNK)],
                        six_v.at[pl.ds(0, CHUNK)])
        for g in range(CHUNK // 16):
            def _lane(i, acc):
                sc_idx = six_v[pl.ds(g * 16 + i, 16)][0]
                sc = inv_v[pl.ds(sc_idx, 16)][0]
                return jnp.where(lanes == i, sc, acc)
            s_v[pl.ds(g * 16, 16)] = lax.fori_loop(
                0, 16, _lane, jnp.zeros((16,), jnp.float32))
        pltpu.sync_copy(s_v, scl_hbm.at[pl.ds(off, CHUNK)])
        return 0
    lax.fori_loop(0, NCHUNK, _extract, 0)


def _make_agg(weighted):
    ncopies = RPT // CHUNK

    @functools.partial(
        pl.kernel,
        out_type=jax.ShapeDtypeStruct((NC, NP, CH), jnp.float32),
        mesh=plsc.VectorSubcoreMesh(**_MESH),
        scratch_types=[
            pltpu.VMEM((CHUNK,), jnp.int32),
            pltpu.VMEM((CHUNK,), jnp.int32),
            pltpu.VMEM((CHUNK + 16,), jnp.int32),
            pltpu.VMEM((CHUNK + 16,), jnp.float32),
            pltpu.VMEM((CHUNK, CH), jnp.float32),
            pltpu.VMEM_SHARED((NP, CH), jnp.float32),
            pltpu.SemaphoreType.DMA,
        ],
    )
    def _agg(h_hbm, gidx_hbm, dst_hbm, et_hbm, scl_hbm, acc_hbm,
             gix_v, dst_v, et_v, s_v, rows_v, acc_sh, sem):
        c = lax.axis_index("c")
        s = lax.axis_index("s")
        wid = s * NC + c

        def _z(i, _):
            for cc in range(CH // 16):
                rows_v[i, pl.ds(cc * 16, 16)] = jnp.zeros((16,), jnp.float32)
            return 0
        lax.fori_loop(0, CHUNK, _z, 0)
        for k in range(ncopies):
            pltpu.sync_copy(rows_v,
                            acc_sh.at[pl.ds(s * RPT + k * CHUNK, CHUNK)])
        plsc.subcore_barrier()

        base = wid * EPW

        def _chunk(k, _):
            off = base + k * CHUNK
            pltpu.sync_copy(gidx_hbm.at[pl.ds(off, CHUNK)], gix_v)
            pltpu.sync_copy(dst_hbm.at[pl.ds(off, CHUNK)], dst_v)
            pltpu.sync_copy(scl_hbm.at[pl.ds(off, CHUNK)],
                            s_v.at[pl.ds(0, CHUNK)])
            if weighted:
                pltpu.sync_copy(et_hbm.at[pl.ds(off, CHUNK)],
                                et_v.at[pl.ds(0, CHUNK)])
            pltpu.async_copy(h_hbm.at[gix_v], rows_v, sem).wait()

            def _scale(e, _):
                sc = s_v[pl.ds(e, 16)][0]
                if weighted:
                    ete = et_v[pl.ds(e, 16)][0]
                for cc in range(CH // 16):
                    if weighted:
                        w = jnp.where(ete == cc, sc, 0.0)
                    else:
                        w = sc
                    rows_v[e, pl.ds(cc * 16, 16)] = (
                        rows_v[e, pl.ds(cc * 16, 16)] * w)
                return 0
            lax.fori_loop(0, CHUNK, _scale, 0)

            pltpu.sync_copy(rows_v, acc_sh.at[dst_v], add=True)
            return 0
        lax.fori_loop(0, NCHUNK, _chunk, 0)

        plsc.subcore_barrier()
        for k in range(ncopies):
            pltpu.sync_copy(acc_sh.at[pl.ds(s * RPT + k * CHUNK, CHUNK)],
                            acc_hbm.at[c, pl.ds(s * RPT + k * CHUNK, CHUNK)])

    return _agg


_agg_l1 = _make_agg(False)
_agg_l2 = _make_agg(True)


def _mm_stacked_body(x_ref, w_ref, o_ref):
    o_ref[0] = jnp.dot(x_ref[...], w_ref[0],
                       preferred_element_type=jnp.float32)


def _combine1_body(x_ref, acc_ref, r1_ref, b1_ref, w2_ref, p2_ref):
    h = jnp.dot(x_ref[...], r1_ref[...], preferred_element_type=jnp.float32)
    h = h + b1_ref[...] + acc_ref[0] + acc_ref[1]
    h = jnp.maximum(h, 0.0)
    p2_ref[...] = jnp.dot(h, w2_ref[...], preferred_element_type=jnp.float32)


def kernel(x, edge_index, edge_type, W1, root1, b1, W2, root2, b2):
    f32 = jnp.float32
    src = edge_index[0].astype(jnp.int32)
    dst = edge_index[1].astype(jnp.int32)
    et = edge_type.astype(jnp.int32)
    gidx = et * N + src
    sidx = et * N + dst

    scl = _sc_scales(sidx)

    h = pl.pallas_call(
        _mm_stacked_body,
        grid=(R, NBLK),
        in_specs=[
            pl.BlockSpec((BLK, CH), lambda r, i: (i, 0)),
            pl.BlockSpec((1, CH, CH), lambda r, i: (r, 0, 0)),
        ],
        out_specs=pl.BlockSpec((1, BLK, CH), lambda r, i: (r, i, 0)),
        out_shape=jax.ShapeDtypeStruct((R, N, CH), f32),
    )(x.astype(f32), W1.astype(f32)).reshape(R * N, CH)
    acc1 = _agg_l1(h, gidx, dst, et, scl)

    w2 = W2.astype(f32)
    cols = [jnp.pad(w2[r], ((0, 0), (0, 16 - OUT))) for r in range(R)]
    cols.append(jnp.pad(root2.astype(f32), ((0, 0), (0, 16 - OUT))))
    w2cat = jnp.concatenate(
        cols + [jnp.zeros((CH, CH - 16 * (R + 1)), f32)], axis=1)

    p2 = pl.pallas_call(
        _combine1_body,
        grid=(NBLK,),
        in_specs=[
            pl.BlockSpec((BLK, CH), lambda i: (i, 0)),
            pl.BlockSpec((NC, BLK, CH), lambda i: (0, i, 0)),
            pl.BlockSpec((CH, CH), lambda i: (0, 0)),
            pl.BlockSpec((1, CH), lambda i: (0, 0)),
            pl.BlockSpec((CH, CH), lambda i: (0, 0)),
        ],
        out_specs=pl.BlockSpec((BLK, CH), lambda i: (i, 0)),
        out_shape=jax.ShapeDtypeStruct((N, CH), f32),
    )(x.astype(f32), acc1, root1.astype(f32),
      b1.astype(f32).reshape(1, CH), w2cat)

    acc2 = _agg_l2(p2, src, dst, et, scl)

    agg = acc2[0, :N] + acc2[1, :N]
    out = p2[:, 16 * R:16 * R + OUT] + b2.astype(f32)
    for r in range(R):
        out = out + agg[:, 16 * r:16 * r + OUT]
    return out


# speedup vs baseline: 10.1750x; 1.0820x over previous
import functools

import jax
import jax.numpy as jnp
from jax import lax
from jax.experimental import pallas as pl
from jax.experimental.pallas import tpu as pltpu
from jax.experimental.pallas import tpu_sc as plsc

N = 10000
E = 320000
R = 4
CH = 128
OUT = 2
CNTP = 40960

NC = 2
NS = 16
NW = NC * NS
EPW = E // NW
CHUNK = 80
NCHUNK = EPW // CHUNK
EPC = E // NS
NCCHUNK = EPC // CHUNK
NP = 10240
RPT = NP // NS
CSTRIPE = CNTP // NS

_MESH = dict(core_axis_name="c", subcore_axis_name="s")

BLK = 400
NBLK = N // BLK


@functools.partial(
    pl.kernel,
    out_type=jax.ShapeDtypeStruct((E,), jnp.float32),
    mesh=plsc.VectorSubcoreMesh(**_MESH),
    scratch_types=[
        pltpu.VMEM((CHUNK + 16,), jnp.int32),
        pltpu.VMEM((CHUNK,), jnp.float32),
        pltpu.VMEM((CHUNK,), jnp.float32),
        pltpu.VMEM((CSTRIPE,), jnp.float32),
        pltpu.VMEM((CNTP + 16,), jnp.float32),
        pltpu.VMEM_SHARED((CNTP,), jnp.float32),
    ],
)
def _sc_scales(sidx_hbm, scl_hbm, six_v, ones_v, s_v, stripe_v, inv_v, cnt_sh):
    c = lax.axis_index("c")
    s = lax.axis_index("s")
    wid = s * NC + c

    def _fill_ones(i, _):
        ones_v[pl.ds(i * 16, 16)] = jnp.full((16,), 1.0, jnp.float32)
        return 0
    lax.fori_loop(0, CHUNK // 16, _fill_ones, 0)

    def _fill_zeros(i, _):
        stripe_v[pl.ds(i * 16, 16)] = jnp.zeros((16,), jnp.float32)
        return 0
    lax.fori_loop(0, CSTRIPE // 16, _fill_zeros, 0)

    pltpu.sync_copy(stripe_v, cnt_sh.at[pl.ds(s * CSTRIPE, CSTRIPE)])
    plsc.subcore_barrier()

    cbase = s * EPC

    def _count(k, _):
        pltpu.sync_copy(sidx_hbm.at[pl.ds(cbase + k * CHUNK, CHUNK)],
                        six_v.at[pl.ds(0, CHUNK)])
        pltpu.sync_copy(ones_v, cnt_sh.at[six_v.at[pl.ds(0, CHUNK)]],
                        add=True)
        return 0
    lax.fori_loop(0, NCCHUNK, _count, 0)
    plsc.subcore_barrier()

    pltpu.sync_copy(cnt_sh.at[pl.ds(s * CSTRIPE, CSTRIPE)], stripe_v)

    def _invb(i, _):
        v = stripe_v[pl.ds(i * 16, 16)]
        stripe_v[pl.ds(i * 16, 16)] = 1.0 / jnp.maximum(v, 1.0)
        return 0
    lax.fori_loop(0, CSTRIPE // 16, _invb, 0)
    pltpu.sync_copy(stripe_v, cnt_sh.at[pl.ds(s * CSTRIPE, CSTRIPE)])
    plsc.subcore_barrier()

    pltpu.sync_copy(cnt_sh, inv_v.at[pl.ds(0, CNTP)])
    lanes = lax.iota(jnp.int32, 16)
    base = wid * EPW

    def _extract(k, _):
        off = base + k * CHUNK
        pltpu.sync_copy(sidx_hbm.at[pl.ds(off, CHUNK)],
                        six_v.at[pl.ds(0, CHUNK)])
        for g in range(CHUNK // 16):
            def _lane(i, acc):
                sc_idx = six_v[pl.ds(g * 16 + i, 16)][0]
                sc = inv_v[pl.ds(sc_idx, 16)][0]
                return jnp.where(lanes == i, sc, acc)
            s_v[pl.ds(g * 16, 16)] = lax.fori_loop(
                0, 16, _lane, jnp.zeros((16,), jnp.float32))
        pltpu.sync_copy(s_v, scl_hbm.at[pl.ds(off, CHUNK)])
        return 0
    lax.fori_loop(0, NCHUNK, _extract, 0)


def _make_agg(weighted):
    ncopies = RPT // CHUNK

    @functools.partial(
        pl.kernel,
        out_type=jax.ShapeDtypeStruct((NC, NP, CH), jnp.float32),
        mesh=plsc.VectorSubcoreMesh(**_MESH),
        scratch_types=[
            pltpu.VMEM((CHUNK,), jnp.int32),
            pltpu.VMEM((CHUNK,), jnp.int32),
            pltpu.VMEM((CHUNK + 16,), jnp.int32),
            pltpu.VMEM((CHUNK + 16,), jnp.float32),
            pltpu.VMEM((CHUNK, CH), jnp.float32),
            pltpu.VMEM_SHARED((NP, CH), jnp.float32),
            pltpu.SemaphoreType.DMA,
        ],
    )
    def _agg(h_hbm, gidx_hbm, dst_hbm, et_hbm, scl_hbm, acc_hbm,
             gix_v, dst_v, et_v, s_v, rows_v, acc_sh, sem):
        c = lax.axis_index("c")
        s = lax.axis_index("s")
        wid = s * NC + c

        def _z(i, _):
            for cc in range(CH // 16):
                rows_v[i, pl.ds(cc * 16, 16)] = jnp.zeros((16,), jnp.float32)
            return 0
        lax.fori_loop(0, CHUNK, _z, 0)
        for k in range(ncopies):
            pltpu.sync_copy(rows_v,
                            acc_sh.at[pl.ds(s * RPT + k * CHUNK, CHUNK)])
        plsc.subcore_barrier()

        base = wid * EPW

        def _chunk(k, _):
            off = base + k * CHUNK
            pltpu.sync_copy(gidx_hbm.at[pl.ds(off, CHUNK)], gix_v)
            pltpu.sync_copy(dst_hbm.at[pl.ds(off, CHUNK)], dst_v)
            pltpu.sync_copy(scl_hbm.at[pl.ds(off, CHUNK)],
                            s_v.at[pl.ds(0, CHUNK)])
            if weighted:
                pltpu.sync_copy(et_hbm.at[pl.ds(off, CHUNK)],
                                et_v.at[pl.ds(0, CHUNK)])
            pltpu.async_copy(h_hbm.at[gix_v], rows_v, sem).wait()

            @plsc.parallel_loop(0, CHUNK, unroll=8)
            def _scale(e):
                sc = s_v[pl.ds(e, 16)][0]
                if weighted:
                    ete = et_v[pl.ds(e, 16)][0]
                for cc in range(CH // 16):
                    if weighted:
                        w = jnp.where(ete == cc, sc, 0.0)
                    else:
                        w = sc
                    rows_v[e, pl.ds(cc * 16, 16)] = (
                        rows_v[e, pl.ds(cc * 16, 16)] * w)

            pltpu.sync_copy(rows_v, acc_sh.at[dst_v], add=True)
            return 0
        lax.fori_loop(0, NCHUNK, _chunk, 0)

        plsc.subcore_barrier()
        for k in range(ncopies):
            pltpu.sync_copy(acc_sh.at[pl.ds(s * RPT + k * CHUNK, CHUNK)],
                            acc_hbm.at[c, pl.ds(s * RPT + k * CHUNK, CHUNK)])

    return _agg


_agg_l1 = _make_agg(False)
_agg_l2 = _make_agg(True)


def _mm_stacked_body(x_ref, w_ref, o_ref):
    o_ref[0] = jnp.dot(x_ref[...], w_ref[0],
                       preferred_element_type=jnp.float32)


def _combine1_body(x_ref, acc_ref, r1_ref, b1_ref, w2_ref, p2_ref):
    h = jnp.dot(x_ref[...], r1_ref[...], preferred_element_type=jnp.float32)
    h = h + b1_ref[...] + acc_ref[0] + acc_ref[1]
    h = jnp.maximum(h, 0.0)
    p2_ref[...] = jnp.dot(h, w2_ref[...], preferred_element_type=jnp.float32)


def kernel(x, edge_index, edge_type, W1, root1, b1, W2, root2, b2):
    f32 = jnp.float32
    src = edge_index[0].astype(jnp.int32)
    dst = edge_index[1].astype(jnp.int32)
    et = edge_type.astype(jnp.int32)
    gidx = et * N + src
    sidx = et * N + dst

    scl = _sc_scales(sidx)

    h = pl.pallas_call(
        _mm_stacked_body,
        grid=(R, NBLK),
        in_specs=[
            pl.BlockSpec((BLK, CH), lambda r, i: (i, 0)),
            pl.BlockSpec((1, CH, CH), lambda r, i: (r, 0, 0)),
        ],
        out_specs=pl.BlockSpec((1, BLK, CH), lambda r, i: (r, i, 0)),
        out_shape=jax.ShapeDtypeStruct((R, N, CH), f32),
    )(x.astype(f32), W1.astype(f32)).reshape(R * N, CH)
    acc1 = _agg_l1(h, gidx, dst, et, scl)

    w2 = W2.astype(f32)
    cols = [jnp.pad(w2[r], ((0, 0), (0, 16 - OUT))) for r in range(R)]
    cols.append(jnp.pad(root2.astype(f32), ((0, 0), (0, 16 - OUT))))
    w2cat = jnp.concatenate(
        cols + [jnp.zeros((CH, CH - 16 * (R + 1)), f32)], axis=1)

    p2 = pl.pallas_call(
        _combine1_body,
        grid=(NBLK,),
        in_specs=[
            pl.BlockSpec((BLK, CH), lambda i: (i, 0)),
            pl.BlockSpec((NC, BLK, CH), lambda i: (0, i, 0)),
            pl.BlockSpec((CH, CH), lambda i: (0, 0)),
            pl.BlockSpec((1, CH), lambda i: (0, 0)),
            pl.BlockSpec((CH, CH), lambda i: (0, 0)),
        ],
        out_specs=pl.BlockSpec((BLK, CH), lambda i: (i, 0)),
        out_shape=jax.ShapeDtypeStruct((N, CH), f32),
    )(x.astype(f32), acc1, root1.astype(f32),
      b1.astype(f32).reshape(1, CH), w2cat)

    acc2 = _agg_l2(p2, src, dst, et, scl)

    agg = acc2[0, :N] + acc2[1, :N]
    out = p2[:, 16 * R:16 * R + OUT] + b2.astype(f32)
    for r in range(R):
        out = out + agg[:, 16 * r:16 * r + OUT]
    return out
